# Initial kernel scaffold; baseline (speedup 1.0000x reference)
import jax
import jax.numpy as jnp
from jax import lax
from jax.experimental import pallas as pl

_B, _D, _H, _W, _K = 4, 16, 512, 512, 8
_P = _H * _W
_CH = 8192
_NCH = _P // _CH
_DELTA_VAR = 1.0
_DELTA_DIST = 2.0
_SQRT_D = 4.0


def _loss_body(data_ref, lab_ref, out_ref):
    b = pl.program_id(0)

    @pl.when(b == 0)
    def _init():
        out_ref[0, 0] = 0.0

    f32 = jnp.float32

    sums_t = jnp.zeros((_K, _D), f32)
    counts = jnp.zeros((_K, 1), f32)
    for i in range(_NCH):
        x = data_ref[0, :, i * _CH:(i + 1) * _CH]
        lab = lab_ref[0, :, i * _CH:(i + 1) * _CH]
        ks = lax.broadcasted_iota(jnp.int32, (_K, _CH), 0)
        oh = (lab == ks).astype(f32)
        sums_t = sums_t + lax.dot_general(
            oh, x, (((1,), (1,)), ((), ())), preferred_element_type=f32)
        counts = counts + jnp.sum(oh, axis=1, keepdims=True)

    centers_t = sums_t / jnp.maximum(counts, 1.0)
    present = counts > 0.0
    n_c = jnp.sum(present.astype(f32))
    cn2 = jnp.sum(centers_t * centers_t, axis=1, keepdims=True)

    var_sum = f32(0.0)
    for i in range(_NCH):
        x = data_ref[0, :, i * _CH:(i + 1) * _CH]
        lab = lab_ref[0, :, i * _CH:(i + 1) * _CH]
        ks = lax.broadcasted_iota(jnp.int32, (_K, _CH), 0)
        oh = (lab == ks).astype(f32)
        s = jnp.sum(x * x, axis=0, keepdims=True)
        proj = lax.dot_general(
            centers_t, x, (((1,), (0,)), ((), ())), preferred_element_type=f32)
        d2 = s + jnp.sum(oh * (cn2 - 2.0 * proj), axis=0, keepdims=True)
        dd = jnp.sqrt(jnp.maximum(d2, 0.0))
        h = jnp.maximum(dd - _DELTA_VAR, 0.0)
        var_sum = var_sum + jnp.sum(h * h)

    g = lax.dot_general(
        centers_t, centers_t, (((1,), (1,)), ((), ())),
        preferred_element_type=f32)
    ones_11 = jnp.ones((1, 1), f32)
    cn2_row = lax.dot_general(
        ones_11, cn2, (((1,), (1,)), ((), ())), preferred_element_type=f32)
    counts_row = lax.dot_general(
        ones_11, counts, (((1,), (1,)), ((), ())), preferred_element_type=f32)
    sq_c = cn2 + cn2_row - 2.0 * g
    ri = lax.broadcasted_iota(jnp.int32, (_K, _K), 0)
    ci = lax.broadcasted_iota(jnp.int32, (_K, _K), 1)
    pair_mask = (ri < ci) & present & (counts_row > 0.0)
    dist = jnp.sqrt(jnp.where(pair_mask, sq_c, 1.0))
    hc = jnp.maximum(2.0 * _DELTA_DIST - dist, 0.0) ** 2
    dist_sum = jnp.sum(jnp.where(pair_mask, hc, 0.0))
    dist_term = dist_sum / jnp.maximum(n_c * (n_c - 1.0), 1.0)

    reg_mask = present & (cn2 > 0.0)
    cn = jnp.sqrt(jnp.where(reg_mask, cn2, 1.0))
    reg_vals = jnp.maximum(cn - _SQRT_D, 0.0)
    reg_term = jnp.sum(jnp.where(reg_mask, reg_vals, 0.0)) / jnp.maximum(n_c, 1.0)

    var_term = var_sum / jnp.maximum(n_c, 1.0)
    total = var_term + dist_term + reg_term
    out_ref[0, 0] += jnp.where(n_c > 1.0, total, 0.0)


def kernel(data, labels):
    data3 = data.reshape(_B, _D, _P)
    lab3 = labels.reshape(_B, 1, _P).astype(jnp.int32)
    out = pl.pallas_call(
        _loss_body,
        grid=(_B,),
        in_specs=[
            pl.BlockSpec((1, _D, _P), lambda b: (b, 0, 0)),
            pl.BlockSpec((1, 1, _P), lambda b: (b, 0, 0)),
        ],
        out_specs=pl.BlockSpec((1, 1), lambda b: (0, 0)),
        out_shape=jax.ShapeDtypeStruct((1, 1), jnp.float32),
    )(data3, lab3)
    return out[0, 0] / jnp.float32(_B)


# speedup vs baseline: 43.7286x; 43.7286x over previous
import jax
import jax.numpy as jnp
from jax import lax
from jax.experimental import pallas as pl

_B, _D, _H, _W, _K = 4, 16, 512, 512, 8
_P = _H * _W
_CH = 8192
_NCH = _P // _CH
_DELTA_VAR = 1.0
_DELTA_DIST = 2.0
_SQRT_D = 4.0


def _loss_body(data_ref, lab_ref, out_ref):
    b = pl.program_id(0)

    @pl.when(b == 0)
    def _init():
        out_ref[...] = jnp.zeros((1, 1), jnp.float32)

    f32 = jnp.float32

    sums_t = jnp.zeros((_K, _D), f32)
    counts = jnp.zeros((_K, 1), f32)
    for i in range(_NCH):
        x = data_ref[0, :, i * _CH:(i + 1) * _CH]
        lab = lab_ref[0, :, i * _CH:(i + 1) * _CH]
        ks = lax.broadcasted_iota(jnp.int32, (_K, _CH), 0)
        oh = (lab == ks).astype(f32)
        sums_t = sums_t + lax.dot_general(
            oh, x, (((1,), (1,)), ((), ())), preferred_element_type=f32)
        counts = counts + jnp.sum(oh, axis=1, keepdims=True)

    centers_t = sums_t / jnp.maximum(counts, 1.0)
    present = counts > 0.0
    n_c = jnp.sum(present.astype(f32))
    cn2 = jnp.sum(centers_t * centers_t, axis=1, keepdims=True)

    var_sum = f32(0.0)
    for i in range(_NCH):
        x = data_ref[0, :, i * _CH:(i + 1) * _CH]
        lab = lab_ref[0, :, i * _CH:(i + 1) * _CH]
        ks = lax.broadcasted_iota(jnp.int32, (_K, _CH), 0)
        oh = (lab == ks).astype(f32)
        s = jnp.sum(x * x, axis=0, keepdims=True)
        proj = lax.dot_general(
            centers_t, x, (((1,), (0,)), ((), ())), preferred_element_type=f32)
        d2 = s + jnp.sum(oh * (cn2 - 2.0 * proj), axis=0, keepdims=True)
        dd = jnp.sqrt(jnp.maximum(d2, 0.0))
        h = jnp.maximum(dd - _DELTA_VAR, 0.0)
        var_sum = var_sum + jnp.sum(h * h)

    g = lax.dot_general(
        centers_t, centers_t, (((1,), (1,)), ((), ())),
        preferred_element_type=f32)
    ones_11 = jnp.ones((1, 1), f32)
    cn2_row = lax.dot_general(
        ones_11, cn2, (((1,), (1,)), ((), ())), preferred_element_type=f32)
    counts_row = lax.dot_general(
        ones_11, counts, (((1,), (1,)), ((), ())), preferred_element_type=f32)
    sq_c = cn2 + cn2_row - 2.0 * g
    ri = lax.broadcasted_iota(jnp.int32, (_K, _K), 0)
    ci = lax.broadcasted_iota(jnp.int32, (_K, _K), 1)
    pair_mask = (ri < ci) & present & (counts_row > 0.0)
    dist = jnp.sqrt(jnp.where(pair_mask, sq_c, 1.0))
    hc = jnp.maximum(2.0 * _DELTA_DIST - dist, 0.0) ** 2
    dist_sum = jnp.sum(jnp.where(pair_mask, hc, 0.0))
    dist_term = dist_sum / jnp.maximum(n_c * (n_c - 1.0), 1.0)

    reg_mask = present & (cn2 > 0.0)
    cn = jnp.sqrt(jnp.where(reg_mask, cn2, 1.0))
    reg_vals = jnp.maximum(cn - _SQRT_D, 0.0)
    reg_term = jnp.sum(jnp.where(reg_mask, reg_vals, 0.0)) / jnp.maximum(n_c, 1.0)

    var_term = var_sum / jnp.maximum(n_c, 1.0)
    total = var_term + dist_term + reg_term
    out_ref[...] += jnp.full((1, 1), jnp.where(n_c > 1.0, total, 0.0))


def kernel(data, labels):
    data3 = data.reshape(_B, _D, _P)
    lab3 = labels.reshape(_B, 1, _P).astype(jnp.int32)
    out = pl.pallas_call(
        _loss_body,
        grid=(_B,),
        in_specs=[
            pl.BlockSpec((1, _D, _P), lambda b: (b, 0, 0)),
            pl.BlockSpec((1, 1, _P), lambda b: (b, 0, 0)),
        ],
        out_specs=pl.BlockSpec((1, 1), lambda b: (0, 0)),
        out_shape=jax.ShapeDtypeStruct((1, 1), jnp.float32),
    )(data3, lab3)
    return out[0, 0] / jnp.float32(_B)
